# baseline (speedup 1.0000x reference)
import functools

import jax
import jax.numpy as jnp
from jax import lax
from jax.experimental import pallas as pl
from jax.experimental.pallas import tpu as pltpu
from jax.experimental.pallas import tpu_sc as plsc

VOCAB = 1000000
EMBED = 128
BATCH = 16384
CTX = 20

NC = 2
NS = 16
L = 16
NW = NC * NS
BPW = BATCH // NW
IW = 128
NCHUNK = BPW // IW
NVEC = EMBED // L
NBUF = 4

_mesh = plsc.VectorSubcoreMesh(core_axis_name="c", subcore_axis_name="s")


@functools.partial(
    pl.kernel,
    mesh=_mesh,
    out_type=jax.ShapeDtypeStruct((BATCH, EMBED), jnp.float32),
    scratch_types=[
        pltpu.VMEM((CTX, NCHUNK, IW), jnp.int32),
        pltpu.VMEM((NBUF, IW, EMBED), jnp.float32),
        pltpu.SemaphoreType.DMA,
        pltpu.SemaphoreType.DMA,
        pltpu.SemaphoreType.DMA,
        pltpu.SemaphoreType.DMA,
    ],
)
def _cbow_sc(ctx_hbm, table_hbm, out_hbm, idx_v, acc_v, sem0, sem1, sem2, sem3):
    wid = lax.axis_index("s") * NC + lax.axis_index("c")
    sems = [sem0, sem1, sem2, sem3]
    pltpu.sync_copy(ctx_hbm.at[wid], idx_v)

    def zero_buf(buf):
        def zrow(b, carry):
            for c in range(NVEC):
                acc_v[buf, b, pl.ds(c * L, L)] = jnp.zeros((L,), jnp.float32)
            return carry

        lax.fori_loop(0, IW, zrow, 0)

    def fire(ck, buf, sem):
        return [
            pltpu.async_copy(
                table_hbm.at[idx_v.at[p, ck]],
                acc_v.at[buf],
                sem,
                add=True,
            )
            for p in range(CTX)
        ]

    pending = {}
    for buf in range(NBUF):
        zero_buf(buf)
    for ck in range(NBUF):
        pending[ck] = fire(ck, ck, sems[ck])

    for ck in range(NCHUNK):
        buf = ck % NBUF
        for cp in pending.pop(ck):
            cp.wait()

        def scale_row(b, carry):
            for c in range(NVEC):
                sl = pl.ds(c * L, L)
                acc_v[buf, b, sl] = acc_v[buf, b, sl] * (1.0 / CTX)
            return carry

        lax.fori_loop(0, IW, scale_row, 0)
        pltpu.sync_copy(
            acc_v.at[buf], out_hbm.at[pl.ds((wid * NCHUNK + ck) * IW, IW)]
        )
        if ck + NBUF < NCHUNK:
            zero_buf(buf)
            pending[ck + NBUF] = fire(ck + NBUF, buf, sems[buf])


def kernel(context, table):
    ctx_arr = (
        context.astype(jnp.int32)
        .reshape(NW, NCHUNK, IW, CTX)
        .transpose(0, 3, 1, 2)
    )
    return _cbow_sc(ctx_arr, table)


# speedup vs baseline: 1.0374x; 1.0374x over previous
import functools

import jax
import jax.numpy as jnp
from jax import lax
from jax.experimental import pallas as pl
from jax.experimental.pallas import tpu as pltpu
from jax.experimental.pallas import tpu_sc as plsc

VOCAB = 1000000
EMBED = 128
BATCH = 16384
CTX = 20

NC = 2
NS = 16
L = 16
NW = NC * NS
BPW = BATCH // NW
IW = 128
NCHUNK = BPW // IW
NVEC = EMBED // L
NBUF = 4

_mesh = plsc.VectorSubcoreMesh(core_axis_name="c", subcore_axis_name="s")


@functools.partial(
    pl.kernel,
    mesh=_mesh,
    out_type=jax.ShapeDtypeStruct((BATCH, EMBED), jnp.float32),
    scratch_types=[
        pltpu.VMEM((NCHUNK, CTX, IW), jnp.int32),
        pltpu.VMEM((NBUF, IW, EMBED), jnp.float32),
        pltpu.SemaphoreType.DMA,
        pltpu.SemaphoreType.DMA,
        pltpu.SemaphoreType.DMA,
        pltpu.SemaphoreType.DMA,
        pltpu.SemaphoreType.DMA,
    ],
)
def _cbow_sc(ctx_hbm, table_hbm, out_hbm, idx_v, acc_v, sem0, sem1, sem2, sem3, osem):
    wid = lax.axis_index("s") * NC + lax.axis_index("c")
    sems = [sem0, sem1, sem2, sem3]
    icps = [
        pltpu.async_copy(
            ctx_hbm.at[:, pl.ds((wid * NCHUNK + k) * IW, IW)],
            idx_v.at[k],
            sems[k],
        )
        for k in range(NCHUNK)
    ]

    def zero_buf(buf):
        def zrow(b, carry):
            for c in range(NVEC):
                acc_v[buf, b, pl.ds(c * L, L)] = jnp.zeros((L,), jnp.float32)
            return carry

        lax.fori_loop(0, IW, zrow, 0)

    def fire(ck, buf, sem):
        return [
            pltpu.async_copy(
                table_hbm.at[idx_v.at[ck, p]],
                acc_v.at[buf],
                sem,
                add=True,
            )
            for p in range(CTX)
        ]

    pending = {}
    for ck in range(NBUF):
        zero_buf(ck)
        icps[ck].wait()
        pending[ck] = fire(ck, ck, sems[ck])

    out_cps = []
    for ck in range(NCHUNK):
        buf = ck % NBUF
        for cp in pending.pop(ck):
            cp.wait()

        def scale_row(b, carry):
            for c in range(NVEC):
                sl = pl.ds(c * L, L)
                acc_v[buf, b, sl] = acc_v[buf, b, sl] * (1.0 / CTX)
            return carry

        lax.fori_loop(0, IW, scale_row, 0)
        out_cps.append(
            pltpu.async_copy(
                acc_v.at[buf],
                out_hbm.at[pl.ds((wid * NCHUNK + ck) * IW, IW)],
                osem,
            )
        )
    for cp in out_cps:
        cp.wait()


def kernel(context, table):
    return _cbow_sc(context.astype(jnp.int32).T, table)
